# baseline (speedup 1.0000x reference)
import functools

import jax
import jax.numpy as jnp
from jax.experimental import pallas as pl

N_EXPERTS = 64
TOP_K = 8
HIDDEN = 2048
ROW_BLOCK = 512


def _gate_kernel(x_ref, w_ref, idx_ref, wgt_ref):
    x = x_ref[...]
    w = w_ref[...]
    logits = jax.lax.dot_general(
        x, w, (((1,), (1,)), ((), ())), preferred_element_type=jnp.float32
    )
    b = logits.shape[0]
    iota = jax.lax.broadcasted_iota(jnp.int32, (b, N_EXPERTS), 1)
    vals = logits
    top_v = []
    top_i = []
    for _ in range(TOP_K):
        m = jnp.max(vals, axis=-1, keepdims=True)
        idx = jnp.min(
            jnp.where(vals >= m, iota, N_EXPERTS), axis=-1, keepdims=True
        )
        top_v.append(m)
        top_i.append(idx)
        vals = jnp.where(iota == idx, -jnp.inf, vals)
    tv = jnp.concatenate(top_v, axis=-1)
    ti = jnp.concatenate(top_i, axis=-1)
    e = jnp.exp(tv - tv[:, 0:1])
    wgt = e / (jnp.sum(e, axis=-1, keepdims=True) + 1e-20)
    idx_ref[...] = ti
    wgt_ref[...] = wgt


@functools.partial(jax.jit, static_argnames=("interpret",))
def _gate(hs2d, weight, interpret=False):
    n = hs2d.shape[0]
    grid = n // ROW_BLOCK
    idx, wgt = pl.pallas_call(
        _gate_kernel,
        grid=(grid,),
        in_specs=[
            pl.BlockSpec((ROW_BLOCK, HIDDEN), lambda i: (i, 0)),
            pl.BlockSpec((N_EXPERTS, HIDDEN), lambda i: (0, 0)),
        ],
        out_specs=[
            pl.BlockSpec((ROW_BLOCK, TOP_K), lambda i: (i, 0)),
            pl.BlockSpec((ROW_BLOCK, TOP_K), lambda i: (i, 0)),
        ],
        out_shape=[
            jax.ShapeDtypeStruct((n, TOP_K), jnp.int32),
            jax.ShapeDtypeStruct((n, TOP_K), jnp.float32),
        ],
        interpret=interpret,
    )(hs2d, weight)
    return idx, wgt


def kernel(hidden_states, weight):
    bsz, seq_len, h = hidden_states.shape
    hs = hidden_states.reshape(-1, h)
    idx, wgt = _gate(hs, weight)
    return (idx, wgt, jnp.float32(0.0))


# speedup vs baseline: 2.2757x; 2.2757x over previous
import functools

import jax
import jax.numpy as jnp
from jax.experimental import pallas as pl

N_EXPERTS = 64
TOP_K = 8
HIDDEN = 2048
ROW_BLOCK = 512


def _gate_kernel(x_ref, w_ref, idx_ref, wgt_ref):
    x = x_ref[...]
    w = w_ref[...]
    logits = jax.lax.dot_general(
        w, x, (((1,), (1,)), ((), ())), preferred_element_type=jnp.float32
    )
    b = logits.shape[1]
    iota = jax.lax.broadcasted_iota(jnp.int32, (N_EXPERTS, b), 0)
    vals = logits
    top_v = []
    top_i = []
    for _ in range(TOP_K):
        m = jnp.max(vals, axis=0, keepdims=True)
        mask = vals >= m
        idx = jnp.min(jnp.where(mask, iota, N_EXPERTS), axis=0, keepdims=True)
        top_v.append(m)
        top_i.append(idx)
        vals = jnp.where(mask, -jnp.inf, vals)
    tv = jnp.concatenate(top_v, axis=0)
    ti = jnp.concatenate(top_i, axis=0)
    e = jnp.exp(tv - tv[0:1])
    wgt = e / (jnp.sum(e, axis=0, keepdims=True) + 1e-20)
    idx_ref[...] = ti
    wgt_ref[...] = wgt


@functools.partial(jax.jit, static_argnames=("interpret",))
def _gate(hs2d, weight, interpret=False):
    n = hs2d.shape[0]
    grid = n // ROW_BLOCK
    idx_t, wgt_t = pl.pallas_call(
        _gate_kernel,
        grid=(grid,),
        in_specs=[
            pl.BlockSpec((ROW_BLOCK, HIDDEN), lambda i: (i, 0)),
            pl.BlockSpec((N_EXPERTS, HIDDEN), lambda i: (0, 0)),
        ],
        out_specs=[
            pl.BlockSpec((TOP_K, ROW_BLOCK), lambda i: (0, i)),
            pl.BlockSpec((TOP_K, ROW_BLOCK), lambda i: (0, i)),
        ],
        out_shape=[
            jax.ShapeDtypeStruct((TOP_K, n), jnp.int32),
            jax.ShapeDtypeStruct((TOP_K, n), jnp.float32),
        ],
        interpret=interpret,
    )(hs2d, weight)
    return idx_t.T, wgt_t.T


def kernel(hidden_states, weight):
    bsz, seq_len, h = hidden_states.shape
    hs = hidden_states.reshape(-1, h)
    idx, wgt = _gate(hs, weight)
    return (idx, wgt, jnp.float32(0.0))


# speedup vs baseline: 2.7440x; 1.2058x over previous
import functools

import jax
import jax.numpy as jnp
from jax.experimental import pallas as pl

N_EXPERTS = 64
TOP_K = 8
HIDDEN = 2048
ROW_BLOCK = 1024


def _gate_kernel(x_ref, w_ref, idx_ref, wgt_ref):
    x = x_ref[...]
    w = w_ref[...]
    logits = jax.lax.dot_general(
        w, x, (((1,), (1,)), ((), ())), preferred_element_type=jnp.float32
    )
    b = logits.shape[1]
    iota = jax.lax.broadcasted_iota(jnp.int32, (N_EXPERTS, b), 0)
    vals = logits
    top_v = []
    top_i = []
    for _ in range(TOP_K):
        m = jnp.max(vals, axis=0, keepdims=True)
        mask = vals >= m
        idx = jnp.min(jnp.where(mask, iota, N_EXPERTS), axis=0, keepdims=True)
        top_v.append(m)
        top_i.append(idx)
        vals = jnp.where(mask, -jnp.inf, vals)
    tv = jnp.concatenate(top_v, axis=0)
    ti = jnp.concatenate(top_i, axis=0)
    e = jnp.exp(tv - tv[0:1])
    wgt = e / (jnp.sum(e, axis=0, keepdims=True) + 1e-20)
    idx_ref[...] = ti
    wgt_ref[...] = wgt


@functools.partial(jax.jit, static_argnames=("interpret",))
def _gate(hs2d, weight, interpret=False):
    n = hs2d.shape[0]
    grid = n // ROW_BLOCK
    idx_t, wgt_t = pl.pallas_call(
        _gate_kernel,
        grid=(grid,),
        in_specs=[
            pl.BlockSpec((ROW_BLOCK, HIDDEN), lambda i: (i, 0)),
            pl.BlockSpec((N_EXPERTS, HIDDEN), lambda i: (0, 0)),
        ],
        out_specs=[
            pl.BlockSpec((TOP_K, ROW_BLOCK), lambda i: (0, i)),
            pl.BlockSpec((TOP_K, ROW_BLOCK), lambda i: (0, i)),
        ],
        out_shape=[
            jax.ShapeDtypeStruct((TOP_K, n), jnp.int32),
            jax.ShapeDtypeStruct((TOP_K, n), jnp.float32),
        ],
        interpret=interpret,
    )(hs2d, weight)
    return idx_t.T, wgt_t.T


def kernel(hidden_states, weight):
    bsz, seq_len, h = hidden_states.shape
    hs = hidden_states.reshape(-1, h)
    idx, wgt = _gate(hs, weight)
    return (idx, wgt, jnp.float32(0.0))


# speedup vs baseline: 2.9876x; 1.0888x over previous
import functools

import jax
import jax.numpy as jnp
from jax.experimental import pallas as pl

N_EXPERTS = 64
TOP_K = 8
HIDDEN = 2048
ROW_BLOCK = 2048


def _gate_kernel(x_ref, w_ref, idx_ref, wgt_ref):
    x = x_ref[...]
    w = w_ref[...]
    logits = jax.lax.dot_general(
        w, x, (((1,), (1,)), ((), ())), preferred_element_type=jnp.float32
    )
    b = logits.shape[1]
    iota = jax.lax.broadcasted_iota(jnp.int32, (N_EXPERTS, b), 0)
    vals = logits
    top_v = []
    top_i = []
    for _ in range(TOP_K):
        m = jnp.max(vals, axis=0, keepdims=True)
        mask = vals >= m
        idx = jnp.min(jnp.where(mask, iota, N_EXPERTS), axis=0, keepdims=True)
        top_v.append(m)
        top_i.append(idx)
        vals = jnp.where(mask, -jnp.inf, vals)
    tv = jnp.concatenate(top_v, axis=0)
    ti = jnp.concatenate(top_i, axis=0)
    e = jnp.exp(tv - tv[0:1])
    wgt = e / (jnp.sum(e, axis=0, keepdims=True) + 1e-20)
    idx_ref[...] = ti
    wgt_ref[...] = wgt


@functools.partial(jax.jit, static_argnames=("interpret",))
def _gate(hs2d, weight, interpret=False):
    n = hs2d.shape[0]
    grid = n // ROW_BLOCK
    idx_t, wgt_t = pl.pallas_call(
        _gate_kernel,
        grid=(grid,),
        in_specs=[
            pl.BlockSpec((ROW_BLOCK, HIDDEN), lambda i: (i, 0)),
            pl.BlockSpec((N_EXPERTS, HIDDEN), lambda i: (0, 0)),
        ],
        out_specs=[
            pl.BlockSpec((TOP_K, ROW_BLOCK), lambda i: (0, i)),
            pl.BlockSpec((TOP_K, ROW_BLOCK), lambda i: (0, i)),
        ],
        out_shape=[
            jax.ShapeDtypeStruct((TOP_K, n), jnp.int32),
            jax.ShapeDtypeStruct((TOP_K, n), jnp.float32),
        ],
        interpret=interpret,
    )(hs2d, weight)
    return idx_t.T, wgt_t.T


def kernel(hidden_states, weight):
    bsz, seq_len, h = hidden_states.shape
    hs = hidden_states.reshape(-1, h)
    idx, wgt = _gate(hs, weight)
    return (idx, wgt, jnp.float32(0.0))


# speedup vs baseline: 2.9927x; 1.0017x over previous
import functools

import jax
import jax.numpy as jnp
from jax.experimental import pallas as pl
from jax.experimental.pallas import tpu as pltpu

N_EXPERTS = 64
TOP_K = 8
HIDDEN = 2048
ROW_BLOCK = 2048


def _gate_kernel(x_ref, w_ref, idx_ref, wgt_ref):
    x = x_ref[...]
    w = w_ref[...]
    logits = jax.lax.dot_general(
        w, x, (((1,), (1,)), ((), ())), preferred_element_type=jnp.float32
    )
    b = logits.shape[1]
    iota = jax.lax.broadcasted_iota(jnp.int32, (N_EXPERTS, b), 0)
    vals = logits
    top_v = []
    top_i = []
    for _ in range(TOP_K):
        m = jnp.max(vals, axis=0, keepdims=True)
        mask = vals >= m
        idx = jnp.min(jnp.where(mask, iota, N_EXPERTS), axis=0, keepdims=True)
        top_v.append(m)
        top_i.append(idx)
        vals = jnp.where(mask, -jnp.inf, vals)
    tv = jnp.concatenate(top_v, axis=0)
    ti = jnp.concatenate(top_i, axis=0)
    e = jnp.exp(tv - tv[0:1])
    wgt = e / (jnp.sum(e, axis=0, keepdims=True) + 1e-20)
    idx_ref[...] = ti
    wgt_ref[...] = wgt


@functools.partial(jax.jit, static_argnames=("interpret",))
def _gate(hs2d, weight, interpret=False):
    n = hs2d.shape[0]
    grid = n // ROW_BLOCK
    idx_t, wgt_t = pl.pallas_call(
        _gate_kernel,
        grid=(grid,),
        in_specs=[
            pl.BlockSpec((ROW_BLOCK, HIDDEN), lambda i: (i, 0)),
            pl.BlockSpec((N_EXPERTS, HIDDEN), lambda i: (0, 0)),
        ],
        out_specs=[
            pl.BlockSpec((TOP_K, ROW_BLOCK), lambda i: (0, i)),
            pl.BlockSpec((TOP_K, ROW_BLOCK), lambda i: (0, i)),
        ],
        out_shape=[
            jax.ShapeDtypeStruct((TOP_K, n), jnp.int32),
            jax.ShapeDtypeStruct((TOP_K, n), jnp.float32),
        ],
        compiler_params=pltpu.CompilerParams(
            dimension_semantics=("parallel",),
        ),
        interpret=interpret,
    )(hs2d, weight)
    return idx_t.T, wgt_t.T


def kernel(hidden_states, weight):
    bsz, seq_len, h = hidden_states.shape
    hs = hidden_states.reshape(-1, h)
    idx, wgt = _gate(hs, weight)
    return (idx, wgt, jnp.float32(0.0))
